# baseline (speedup 1.0000x reference)
import functools

import jax
import jax.numpy as jnp
from jax import lax
from jax.experimental import pallas as pl
from jax.experimental.pallas import tpu as pltpu
from jax.experimental.pallas import tpu_sc as plsc

N_NODES = 10000
N_EDGES = 320000
D = 128
NC = 2
NS = 16
L = 16
NW = NC * NS
CB = 128
NCHUNK = N_EDGES // CB
TRIPS = -(-NCHUNK // NW)
RPT = N_NODES // NS
ROW_BLK = 2000


def _mm_body(h_ref, n_ref, w_ref, o_ref):
    o_ref[...] = jnp.dot(h_ref[...] * n_ref[...], w_ref[...],
                         preferred_element_type=jnp.float32)


def _fin_body(p_ref, n_ref, b_ref, o_ref):
    s = p_ref[0] + p_ref[1]
    o_ref[...] = jnp.maximum(s * n_ref[...] + b_ref[...], 0.0)


_sc_mesh = plsc.VectorSubcoreMesh(core_axis_name="c", subcore_axis_name="s")


@functools.partial(
    pl.kernel,
    out_type=jax.ShapeDtypeStruct((NC, N_NODES, D), jnp.float32),
    mesh=_sc_mesh,
    scratch_types=[
        pltpu.VMEM((3, 2, CB), jnp.int32),
        pltpu.VMEM((CB, D), jnp.float32),
        pltpu.VMEM((CB, D), jnp.float32),
        pltpu.VMEM((CB, D), jnp.float32),
        pltpu.VMEM_SHARED((N_NODES, D), jnp.float32),
        pltpu.SemaphoreType.DMA,
        pltpu.SemaphoreType.DMA,
        pltpu.SemaphoreType.DMA,
        pltpu.SemaphoreType.DMA,
        pltpu.SemaphoreType.DMA,
        pltpu.SemaphoreType.DMA,
        pltpu.SemaphoreType.DMA,
        pltpu.SemaphoreType.DMA,
        pltpu.SemaphoreType.DMA,
    ],
)
def _sc_scatter(x_hbm, ei_hbm, z_hbm, out_hbm, ibuf, rows_a, rows_b, rows_c,
                agg_sh, gsem_a, gsem_b, gsem_c, gsem_d, gsem_e, gsem_f,
                isem_a, isem_b, isem_c):
    cid = lax.axis_index("c")
    sid = lax.axis_index("s")
    wid = cid * NS + sid
    rows = (rows_a, rows_b, rows_c)
    gsems = ((gsem_a, gsem_b), (gsem_c, gsem_d), (gsem_e, gsem_f))
    isems = (isem_a, isem_b, isem_c)

    def start_idx(c, p):
        pltpu.async_copy(ei_hbm.at[:, pl.ds(c * CB, CB)], ibuf.at[p],
                         isems[p])

    def wait_idx(c, p):
        pltpu.make_async_copy(ei_hbm.at[:, pl.ds(c * CB, CB)], ibuf.at[p],
                              isems[p]).wait()

    def start_gather(p):
        pltpu.async_copy(x_hbm.at[ibuf.at[p, 0]], rows[p], gsems[p][0])

    def wait_gather(p):
        pltpu.make_async_copy(x_hbm.at[ibuf.at[p, 0]], rows[p],
                              gsems[p][0]).wait()

    start_idx(wid, 0)
    wait_idx(wid, 0)
    start_gather(0)

    @pl.when(wid + NW < NCHUNK)
    def _prime_idx1():
        start_idx(wid + NW, 1)

    @pl.when(wid + 2 * NW < NCHUNK)
    def _prime_idx2():
        start_idx(wid + 2 * NW, 2)

    @pl.when(sid < NS - 1)
    def _zero_agg():
        pltpu.sync_copy(z_hbm.at[pl.ds(0, 624)],
                        agg_sh.at[pl.ds(sid * 624, 624)])

    @pl.when(sid == NS - 1)
    def _zero_agg_last():
        pltpu.sync_copy(z_hbm, agg_sh.at[pl.ds(9360, 640)])
    plsc.subcore_barrier()

    @pl.loop(0, TRIPS + (-TRIPS) % 3, step=3)
    def _edges(i):
        for p in range(3):
            c = wid + (i + p) * NW
            pn = (p + 1) % 3

            @pl.when(c < NCHUNK)
            def _chunk():
                @pl.when(c + NW < NCHUNK)
                def _next_gather():
                    wait_idx(c + NW, pn)
                    start_gather(pn)

                wait_gather(p)
                pltpu.sync_copy(rows[p], agg_sh.at[ibuf.at[p, 1]], add=True)

                @pl.when(c + 3 * NW < NCHUNK)
                def _next_idx():
                    start_idx(c + 3 * NW, p)

    plsc.subcore_barrier()

    @pl.when(sid < NS - 1)
    def _copy_out():
        pltpu.sync_copy(agg_sh.at[pl.ds(sid * 624, 624)],
                        out_hbm.at[cid, pl.ds(sid * 624, 624)])

    @pl.when(sid == NS - 1)
    def _copy_out_last():
        pltpu.sync_copy(agg_sh.at[pl.ds(9360, 640)],
                        out_hbm.at[cid, pl.ds(9360, 640)])


@jax.jit
def kernel(h, W, b, norm, edge_index):
    x = pl.pallas_call(
        _mm_body,
        grid=(N_NODES // ROW_BLK,),
        in_specs=[
            pl.BlockSpec((ROW_BLK, D), lambda i: (i, 0)),
            pl.BlockSpec((ROW_BLK, 1), lambda i: (i, 0)),
            pl.BlockSpec((D, D), lambda i: (0, 0)),
        ],
        out_specs=pl.BlockSpec((ROW_BLK, D), lambda i: (i, 0)),
        out_shape=jax.ShapeDtypeStruct((N_NODES, D), jnp.float32),
    )(h, norm, W)

    parts = _sc_scatter(x, edge_index, jnp.zeros((640, D), jnp.float32))

    out = pl.pallas_call(
        _fin_body,
        grid=(N_NODES // ROW_BLK,),
        in_specs=[
            pl.BlockSpec((NC, ROW_BLK, D), lambda i: (0, i, 0)),
            pl.BlockSpec((ROW_BLK, 1), lambda i: (i, 0)),
            pl.BlockSpec((1, D), lambda i: (0, 0)),
        ],
        out_specs=pl.BlockSpec((ROW_BLK, D), lambda i: (i, 0)),
        out_shape=jax.ShapeDtypeStruct((N_NODES, D), jnp.float32),
    )(parts, norm, b.reshape(1, D))
    return out


# speedup vs baseline: 1.0258x; 1.0258x over previous
import functools

import jax
import jax.numpy as jnp
from jax import lax
from jax.experimental import pallas as pl
from jax.experimental.pallas import tpu as pltpu
from jax.experimental.pallas import tpu_sc as plsc

N_NODES = 10000
N_EDGES = 320000
D = 128
NC = 2
NS = 16
L = 16
NW = NC * NS
CB = 128
NCHUNK = N_EDGES // CB
TRIPS = -(-NCHUNK // NW)
RPT = N_NODES // NS
ROW_BLK = 2000


def _mm_body(h_ref, n_ref, w_ref, o_ref):
    o_ref[...] = jnp.dot(h_ref[...] * n_ref[...], w_ref[...],
                         preferred_element_type=jnp.float32)


def _fin_body(p_ref, n_ref, b_ref, o_ref):
    s = p_ref[0] + p_ref[1]
    o_ref[...] = jnp.maximum(s * n_ref[...] + b_ref[...], 0.0)


_sc_mesh = plsc.VectorSubcoreMesh(core_axis_name="c", subcore_axis_name="s")


@functools.partial(
    pl.kernel,
    out_type=jax.ShapeDtypeStruct((NC, N_NODES, D), jnp.float32),
    mesh=_sc_mesh,
    scratch_types=[
        pltpu.VMEM((3, 2, CB), jnp.int32),
        pltpu.VMEM((CB, D), jnp.float32),
        pltpu.VMEM((CB, D), jnp.float32),
        pltpu.VMEM((CB, D), jnp.float32),
        pltpu.VMEM_SHARED((N_NODES, D), jnp.float32),
        pltpu.SemaphoreType.DMA,
        pltpu.SemaphoreType.DMA,
        pltpu.SemaphoreType.DMA,
        pltpu.SemaphoreType.DMA,
        pltpu.SemaphoreType.DMA,
        pltpu.SemaphoreType.DMA,
        pltpu.SemaphoreType.DMA,
        pltpu.SemaphoreType.DMA,
        pltpu.SemaphoreType.DMA,
    ],
)
def _sc_scatter(x_hbm, ei_hbm, out_hbm, ibuf, rows_a, rows_b, rows_c, agg_sh,
                gsem_a, gsem_b, gsem_c, gsem_d, gsem_e, gsem_f,
                isem_a, isem_b, isem_c):
    cid = lax.axis_index("c")
    sid = lax.axis_index("s")
    wid = cid * NS + sid
    rows = (rows_a, rows_b, rows_c)
    gsems = ((gsem_a, gsem_b), (gsem_c, gsem_d), (gsem_e, gsem_f))
    isems = (isem_a, isem_b, isem_c)

    zeros = jnp.zeros((L,), jnp.float32)

    @pl.loop(0, CB)
    def _zero_rows(i):
        for c in range(D // L):
            rows_a[i, pl.ds(c * L, L)] = zeros

    for k in range(RPT // CB):
        pltpu.sync_copy(rows_a, agg_sh.at[pl.ds(sid * RPT + k * CB, CB)])
    if RPT % CB:
        pltpu.sync_copy(
            rows_a.at[pl.ds(0, RPT % CB)],
            agg_sh.at[pl.ds(sid * RPT + (RPT // CB) * CB, RPT % CB)])
    plsc.subcore_barrier()

    def start_idx(c, p):
        pltpu.async_copy(ei_hbm.at[:, pl.ds(c * CB, CB)], ibuf.at[p],
                         isems[p])

    def wait_idx(c, p):
        pltpu.make_async_copy(ei_hbm.at[:, pl.ds(c * CB, CB)], ibuf.at[p],
                              isems[p]).wait()

    def start_gather(p):
        pltpu.async_copy(x_hbm.at[ibuf.at[p, 0]], rows[p], gsems[p][0])

    def wait_gather(p):
        pltpu.make_async_copy(x_hbm.at[ibuf.at[p, 0]], rows[p],
                              gsems[p][0]).wait()

    start_idx(wid, 0)
    wait_idx(wid, 0)
    start_gather(0)

    @pl.when(wid + NW < NCHUNK)
    def _prime_idx1():
        start_idx(wid + NW, 1)

    @pl.when(wid + 2 * NW < NCHUNK)
    def _prime_idx2():
        start_idx(wid + 2 * NW, 2)

    @pl.loop(0, TRIPS + (-TRIPS) % 3, step=3)
    def _edges(i):
        for p in range(3):
            c = wid + (i + p) * NW
            pn = (p + 1) % 3

            @pl.when(c < NCHUNK)
            def _chunk():
                @pl.when(c + NW < NCHUNK)
                def _next_gather():
                    wait_idx(c + NW, pn)
                    start_gather(pn)

                wait_gather(p)
                pltpu.sync_copy(rows[p], agg_sh.at[ibuf.at[p, 1]], add=True)

                @pl.when(c + 3 * NW < NCHUNK)
                def _next_idx():
                    start_idx(c + 3 * NW, p)

    plsc.subcore_barrier()

    @pl.when(sid < NS - 1)
    def _copy_out():
        pltpu.sync_copy(agg_sh.at[pl.ds(sid * 624, 624)],
                        out_hbm.at[cid, pl.ds(sid * 624, 624)])

    @pl.when(sid == NS - 1)
    def _copy_out_last():
        pltpu.sync_copy(agg_sh.at[pl.ds(9360, 640)],
                        out_hbm.at[cid, pl.ds(9360, 640)])


@jax.jit
def kernel(h, W, b, norm, edge_index):
    x = pl.pallas_call(
        _mm_body,
        grid=(N_NODES // ROW_BLK,),
        in_specs=[
            pl.BlockSpec((ROW_BLK, D), lambda i: (i, 0)),
            pl.BlockSpec((ROW_BLK, 1), lambda i: (i, 0)),
            pl.BlockSpec((D, D), lambda i: (0, 0)),
        ],
        out_specs=pl.BlockSpec((ROW_BLK, D), lambda i: (i, 0)),
        out_shape=jax.ShapeDtypeStruct((N_NODES, D), jnp.float32),
    )(h, norm, W)

    parts = _sc_scatter(x, edge_index)

    out = pl.pallas_call(
        _fin_body,
        grid=(N_NODES // ROW_BLK,),
        in_specs=[
            pl.BlockSpec((NC, ROW_BLK, D), lambda i: (0, i, 0)),
            pl.BlockSpec((ROW_BLK, 1), lambda i: (i, 0)),
            pl.BlockSpec((1, D), lambda i: (0, 0)),
        ],
        out_specs=pl.BlockSpec((ROW_BLK, D), lambda i: (i, 0)),
        out_shape=jax.ShapeDtypeStruct((N_NODES, D), jnp.float32),
    )(parts, norm, b.reshape(1, D))
    return out
